# baseline (speedup 1.0000x reference)
import functools

import jax
import jax.numpy as jnp
from jax import lax
from jax.experimental import pallas as pl
from jax.experimental.pallas import tpu as pltpu
from jax.experimental.pallas import tpu_sc as plsc

N, D, E, G, DF, TOPK = 2048, 768, 8, 2, 1536, 2
EG = E // G
BLK = 128
M = N * TOPK + E * BLK
NB = M // BLK
ASSIGN = N * TOPK
NWORK = 32
APW = ASSIGN // NWORK
TPW = N // NWORK
DH = D // 2


def _router_body(x_ref, wgg_ref, wge_ref,
                 pos0_ref, pos1_ref, g0_ref, g1_ref, be_ref, valid_ref):
    x = x_ref[...]
    gl = jnp.dot(x, wgg_ref[...], preferred_element_type=jnp.float32)
    el = jnp.dot(x, wge_ref[...], preferred_element_type=jnp.float32)
    p_group = jax.nn.softmax(gl, axis=-1)
    parts = []
    for g in range(G):
        sm = jax.nn.softmax(el[:, g * EG:(g + 1) * EG], axis=-1)
        parts.append(sm * p_group[:, g:g + 1])
    probs = jnp.concatenate(parts, axis=1)

    iota = lax.broadcasted_iota(jnp.int32, (N, E), 1)
    v0 = jnp.max(probs, axis=1, keepdims=True)
    i0 = jnp.min(jnp.where(probs == v0, iota, E), axis=1, keepdims=True)
    masked = jnp.where(iota == i0, -1.0, probs)
    v1 = jnp.max(masked, axis=1, keepdims=True)
    i1 = jnp.min(jnp.where(masked == v1, iota, E), axis=1, keepdims=True)
    s = v0 + v1 + 1e-9
    g0_ref[...] = v0 / s
    g1_ref[...] = v1 / s

    oh0 = (iota == i0).astype(jnp.float32)
    oh1 = (iota == i1).astype(jnp.float32)
    oh = oh0 + oh1
    cs = N // 16
    tri_r = lax.broadcasted_iota(jnp.int32, (cs, cs), 0)
    tri_c = lax.broadcasted_iota(jnp.int32, (cs, cs), 1)
    tri = (tri_r > tri_c).astype(jnp.float32)
    cum_parts = []
    run = jnp.zeros((1, E), jnp.float32)
    for c in range(16):
        blk = oh[c * cs:(c + 1) * cs]
        local = jnp.dot(tri, blk, preferred_element_type=jnp.float32)
        cum_parts.append(local + run)
        run = run + jnp.sum(blk, axis=0, keepdims=True)
    cum = jnp.concatenate(cum_parts, axis=0)
    rank0 = jnp.sum(cum * oh0, axis=1, keepdims=True)
    rank1 = jnp.sum(cum * oh1, axis=1, keepdims=True)

    counts = run
    padded = jnp.floor((counts + (BLK - 1)) * (1.0 / BLK)) * BLK
    er = lax.broadcasted_iota(jnp.int32, (E, E), 0)
    ec = lax.broadcasted_iota(jnp.int32, (E, E), 1)
    incl = (er <= ec).astype(jnp.float32)
    seg_end = jnp.dot(padded, incl, preferred_element_type=jnp.float32)
    offsets = seg_end - padded

    pos0 = jnp.sum(oh0 * offsets, axis=1, keepdims=True) + rank0
    pos1 = jnp.sum(oh1 * offsets, axis=1, keepdims=True) + rank1
    pos0_ref[...] = pos0.astype(jnp.int32)
    pos1_ref[...] = pos1.astype(jnp.int32)

    starts = (lax.broadcasted_iota(jnp.int32, (1, NB), 1) * BLK).astype(jnp.float32)
    be = jnp.zeros((1, NB), jnp.int32)
    for e in range(E):
        be = be + (starts >= seg_end[0, e]).astype(jnp.int32)
    be_ref[...] = jnp.minimum(be, E - 1)
    valid_ref[...] = (starts < seg_end[0, E - 1]).astype(jnp.int32)


def _router(x, Wg_group, Wg_expert):
    return pl.pallas_call(
        _router_body,
        out_shape=(
            jax.ShapeDtypeStruct((N, 1), jnp.int32),
            jax.ShapeDtypeStruct((N, 1), jnp.int32),
            jax.ShapeDtypeStruct((N, 1), jnp.float32),
            jax.ShapeDtypeStruct((N, 1), jnp.float32),
            jax.ShapeDtypeStruct((1, NB), jnp.int32),
            jax.ShapeDtypeStruct((1, NB), jnp.int32),
        ),
    )(x, Wg_group, Wg_expert)


_SC_MESH = plsc.VectorSubcoreMesh(core_axis_name="c", subcore_axis_name="s")


@functools.partial(
    pl.kernel,
    out_type=jax.ShapeDtypeStruct((M, DH), jnp.float32),
    mesh=_SC_MESH,
    scratch_types=[
        pltpu.VMEM((APW,), jnp.int32),
        pltpu.VMEM((APW,), jnp.int32),
        pltpu.VMEM((APW, DH), jnp.float32),
        pltpu.SemaphoreType.DMA,
    ],
)
def _dispatch(x_hbm, p0_hbm, p1_hbm, xs_hbm, pos_v, tok_v, rows_v, sem):
    wid = lax.axis_index("s") * 2 + lax.axis_index("c")
    tbase = lax.bitwise_and(wid, NWORK // 2 - 1) * APW

    @pl.when(wid < NWORK // 2)
    def _():
        pltpu.sync_copy(p0_hbm.at[pl.ds(tbase, APW)], pos_v)

    @pl.when(wid >= NWORK // 2)
    def _():
        pltpu.sync_copy(p1_hbm.at[pl.ds(tbase, APW)], pos_v)

    for j in range(APW // 16):
        tok_v[pl.ds(j * 16, 16)] = lax.iota(jnp.int32, 16) + (tbase + j * 16)
    pltpu.async_copy(x_hbm.at[tok_v], rows_v, sem).wait()
    pltpu.async_copy(rows_v, xs_hbm.at[pos_v], sem).wait()


def _w2r_body(w2_ref, b2_ref, wrt_ref, br_ref, w2r_ref, b2r_ref):
    w2r_ref[0] = lax.dot_general(wrt_ref[...], w2_ref[0],
                                 (((1,), (1,)), ((), ())))
    b2r = jnp.sum(b2_ref[0] * wrt_ref[...]) + jnp.sum(br_ref[...])
    b2r_ref[...] = b2r.reshape(1, 1, 1)


def _w2r(W2, b2, wrT, b_risk):
    return pl.pallas_call(
        _w2r_body,
        grid=(E,),
        in_specs=[
            pl.BlockSpec((1, DF, D), lambda e: (e, 0, 0)),
            pl.BlockSpec((1, 1, D), lambda e: (e, 0, 0)),
            pl.BlockSpec((1, D), lambda e: (0, 0)),
            pl.BlockSpec((1, 1), lambda e: (0, 0)),
        ],
        out_specs=(
            pl.BlockSpec((1, 1, DF), lambda e: (e, 0, 0)),
            pl.BlockSpec((1, 1, 1), lambda e: (e, 0, 0)),
        ),
        out_shape=(
            jax.ShapeDtypeStruct((E, 1, DF), jnp.float32),
            jax.ShapeDtypeStruct((E, 1, 1), jnp.float32),
        ),
    )(W2, b2.reshape(E, 1, D), wrT, b_risk.reshape(1, 1))


def _ffn_body(be_ref, valid_ref, xs_ref, w1_ref, b1_ref, w2r_ref, b2r_ref,
              out_ref):
    i = pl.program_id(0)

    @pl.when(valid_ref[0, i] > 0)
    def _():
        xb = xs_ref[...]
        h = jnp.dot(xb, w1_ref[0], preferred_element_type=jnp.float32)
        h = jax.nn.gelu(h + b1_ref[0])
        z = jnp.sum(h * w2r_ref[0], axis=1) + jnp.sum(b2r_ref[...])
        out_ref[...] = z[:, None]


def _ffn(block_expert, valid, xs, W1, b1, w2r, b2r):
    grid_spec = pltpu.PrefetchScalarGridSpec(
        num_scalar_prefetch=2,
        grid=(NB,),
        in_specs=[
            pl.BlockSpec((BLK, D), lambda i, be, va: (i, 0)),
            pl.BlockSpec((1, D, DF), lambda i, be, va: (be[0, i], 0, 0)),
            pl.BlockSpec((1, 1, DF), lambda i, be, va: (be[0, i], 0, 0)),
            pl.BlockSpec((1, 1, DF), lambda i, be, va: (be[0, i], 0, 0)),
            pl.BlockSpec((1, 1, 1), lambda i, be, va: (be[0, i], 0, 0)),
        ],
        out_specs=pl.BlockSpec((BLK, 1), lambda i, be, va: (i, 0)),
    )
    return pl.pallas_call(
        _ffn_body,
        grid_spec=grid_spec,
        out_shape=jax.ShapeDtypeStruct((M, 1), jnp.float32),
    )(block_expert, valid, xs, W1, b1.reshape(E, 1, DF), w2r, b2r)


@functools.partial(
    pl.kernel,
    out_type=jax.ShapeDtypeStruct((N,), jnp.float32),
    mesh=_SC_MESH,
    scratch_types=[
        pltpu.VMEM((TPW,), jnp.int32),
        pltpu.VMEM((TPW,), jnp.int32),
        pltpu.VMEM((TPW,), jnp.float32),
        pltpu.VMEM((TPW,), jnp.float32),
        pltpu.VMEM((TPW,), jnp.float32),
        pltpu.VMEM((TPW,), jnp.float32),
        pltpu.VMEM((TPW,), jnp.float32),
        pltpu.SemaphoreType.DMA,
    ],
)
def _combine(z_hbm, p0_hbm, p1_hbm, g0_hbm, g1_hbm, out_hbm,
             p0_v, p1_v, z0_v, z1_v, g0_v, g1_v, r_v, sem):
    wid = lax.axis_index("s") * 2 + lax.axis_index("c")
    base = wid * TPW
    pltpu.sync_copy(p0_hbm.at[pl.ds(base, TPW)], p0_v)
    pltpu.sync_copy(p1_hbm.at[pl.ds(base, TPW)], p1_v)
    pltpu.sync_copy(g0_hbm.at[pl.ds(base, TPW)], g0_v)
    pltpu.sync_copy(g1_hbm.at[pl.ds(base, TPW)], g1_v)
    pltpu.async_copy(z_hbm.at[p0_v], z0_v, sem).wait()
    pltpu.async_copy(z_hbm.at[p1_v], z1_v, sem).wait()
    for j in range(TPW // 16):
        sl = pl.ds(j * 16, 16)
        r_v[sl] = g0_v[sl] * z0_v[sl] + g1_v[sl] * z1_v[sl]
    pltpu.sync_copy(r_v, out_hbm.at[pl.ds(base, TPW)])


def kernel(x, Wg_group, Wg_expert, W1, b1, W2, b2, W_risk, b_risk):
    pos0, pos1, g0, g1, block_expert, valid = _router(x, Wg_group, Wg_expert)
    p0, p1 = pos0.reshape(N), pos1.reshape(N)
    w2r, b2r = _w2r(W2, b2, W_risk.reshape(1, D), b_risk)
    x_pk = lax.bitcast_convert_type(
        x.astype(jnp.bfloat16).reshape(N, DH, 2), jnp.float32)
    xs_pk = _dispatch(x_pk, p0, p1)
    xs = lax.bitcast_convert_type(xs_pk, jnp.bfloat16).reshape(M, D)
    z = _ffn(block_expert, valid, xs, W1.astype(jnp.bfloat16), b1, w2r, b2r)
    risk = _combine(z.reshape(M), p0, p1, g0.reshape(N), g1.reshape(N))
    return risk


# speedup vs baseline: 2.0412x; 2.0412x over previous
import functools

import jax
import jax.numpy as jnp
from jax import lax
from jax.experimental import pallas as pl
from jax.experimental.pallas import tpu as pltpu
from jax.experimental.pallas import tpu_sc as plsc

N, D, E, G, DF, TOPK = 2048, 768, 8, 2, 1536, 2
EG = E // G
BLK = 128
M = N * TOPK + E * BLK
NB = M // BLK
ASSIGN = N * TOPK
NWORK = 32
APW = ASSIGN // NWORK
TPW = N // NWORK
DH = D // 2


def _router_body(x_ref, wgg_ref, wge_ref,
                 pos0_ref, pos1_ref, g0_ref, g1_ref, be_ref, valid_ref):
    x = x_ref[...]
    gl = jnp.dot(x, wgg_ref[...], preferred_element_type=jnp.float32)
    el = jnp.dot(x, wge_ref[...], preferred_element_type=jnp.float32)
    p_group = jax.nn.softmax(gl, axis=-1)
    parts = []
    for g in range(G):
        sm = jax.nn.softmax(el[:, g * EG:(g + 1) * EG], axis=-1)
        parts.append(sm * p_group[:, g:g + 1])
    probs = jnp.concatenate(parts, axis=1)

    iota = lax.broadcasted_iota(jnp.int32, (N, E), 1)
    v0 = jnp.max(probs, axis=1, keepdims=True)
    i0 = jnp.min(jnp.where(probs == v0, iota, E), axis=1, keepdims=True)
    masked = jnp.where(iota == i0, -1.0, probs)
    v1 = jnp.max(masked, axis=1, keepdims=True)
    i1 = jnp.min(jnp.where(masked == v1, iota, E), axis=1, keepdims=True)
    s = v0 + v1 + 1e-9
    g0_ref[...] = v0 / s
    g1_ref[...] = v1 / s

    oh0 = (iota == i0).astype(jnp.float32)
    oh1 = (iota == i1).astype(jnp.float32)
    oh = oh0 + oh1
    cs = N // 16
    tri_r = lax.broadcasted_iota(jnp.int32, (cs, cs), 0)
    tri_c = lax.broadcasted_iota(jnp.int32, (cs, cs), 1)
    tri = (tri_r > tri_c).astype(jnp.float32)
    cum_parts = []
    run = jnp.zeros((1, E), jnp.float32)
    for c in range(16):
        blk = oh[c * cs:(c + 1) * cs]
        local = jnp.dot(tri, blk, preferred_element_type=jnp.float32)
        cum_parts.append(local + run)
        run = run + jnp.sum(blk, axis=0, keepdims=True)
    cum = jnp.concatenate(cum_parts, axis=0)
    rank0 = jnp.sum(cum * oh0, axis=1, keepdims=True)
    rank1 = jnp.sum(cum * oh1, axis=1, keepdims=True)

    counts = run
    padded = jnp.floor((counts + (BLK - 1)) * (1.0 / BLK)) * BLK
    er = lax.broadcasted_iota(jnp.int32, (E, E), 0)
    ec = lax.broadcasted_iota(jnp.int32, (E, E), 1)
    incl = (er <= ec).astype(jnp.float32)
    seg_end = jnp.dot(padded, incl, preferred_element_type=jnp.float32)
    offsets = seg_end - padded

    pos0 = jnp.sum(oh0 * offsets, axis=1, keepdims=True) + rank0
    pos1 = jnp.sum(oh1 * offsets, axis=1, keepdims=True) + rank1
    pos0_ref[...] = pos0.astype(jnp.int32)
    pos1_ref[...] = pos1.astype(jnp.int32)

    starts = (lax.broadcasted_iota(jnp.int32, (1, NB), 1) * BLK).astype(jnp.float32)
    be = jnp.zeros((1, NB), jnp.int32)
    for e in range(E):
        be = be + (starts >= seg_end[0, e]).astype(jnp.int32)
    be_ref[...] = jnp.minimum(be, E - 1)
    valid_ref[...] = (starts < seg_end[0, E - 1]).astype(jnp.int32)


def _router(x, Wg_group, Wg_expert):
    return pl.pallas_call(
        _router_body,
        out_shape=(
            jax.ShapeDtypeStruct((N, 1), jnp.int32),
            jax.ShapeDtypeStruct((N, 1), jnp.int32),
            jax.ShapeDtypeStruct((N, 1), jnp.float32),
            jax.ShapeDtypeStruct((N, 1), jnp.float32),
            jax.ShapeDtypeStruct((1, NB), jnp.int32),
            jax.ShapeDtypeStruct((1, NB), jnp.int32),
        ),
    )(x, Wg_group, Wg_expert)


_SC_MESH = plsc.VectorSubcoreMesh(core_axis_name="c", subcore_axis_name="s")


@functools.partial(
    pl.kernel,
    out_type=jax.ShapeDtypeStruct((M, D), jnp.float32),
    mesh=_SC_MESH,
    scratch_types=[
        pltpu.VMEM((APW,), jnp.int32),
        pltpu.VMEM((APW,), jnp.int32),
        pltpu.VMEM((APW, D), jnp.float32),
        pltpu.SemaphoreType.DMA,
    ],
)
def _dispatch(x_hbm, p0_hbm, p1_hbm, xs_hbm, pos_v, tok_v, rows_v, sem):
    wid = lax.axis_index("s") * 2 + lax.axis_index("c")
    tbase = lax.bitwise_and(wid, NWORK // 2 - 1) * APW

    @pl.when(wid < NWORK // 2)
    def _():
        pltpu.sync_copy(p0_hbm.at[pl.ds(tbase, APW)], pos_v)

    @pl.when(wid >= NWORK // 2)
    def _():
        pltpu.sync_copy(p1_hbm.at[pl.ds(tbase, APW)], pos_v)

    for j in range(APW // 16):
        tok_v[pl.ds(j * 16, 16)] = lax.iota(jnp.int32, 16) + (tbase + j * 16)
    pltpu.async_copy(x_hbm.at[tok_v], rows_v, sem).wait()
    pltpu.async_copy(rows_v, xs_hbm.at[pos_v], sem).wait()


def _w2r_body(w2_ref, b2_ref, wrt_ref, br_ref, w2r_ref, b2r_ref):
    w2r_ref[0] = lax.dot_general(wrt_ref[...], w2_ref[0],
                                 (((1,), (1,)), ((), ())))
    b2r = jnp.sum(b2_ref[0] * wrt_ref[...]) + jnp.sum(br_ref[...])
    b2r_ref[...] = b2r.reshape(1, 1, 1)


def _w2r(W2, b2, wrT, b_risk):
    return pl.pallas_call(
        _w2r_body,
        grid=(E,),
        in_specs=[
            pl.BlockSpec((1, DF, D), lambda e: (e, 0, 0)),
            pl.BlockSpec((1, 1, D), lambda e: (e, 0, 0)),
            pl.BlockSpec((1, D), lambda e: (0, 0)),
            pl.BlockSpec((1, 1), lambda e: (0, 0)),
        ],
        out_specs=(
            pl.BlockSpec((1, 1, DF), lambda e: (e, 0, 0)),
            pl.BlockSpec((1, 1, 1), lambda e: (e, 0, 0)),
        ),
        out_shape=(
            jax.ShapeDtypeStruct((E, 1, DF), jnp.float32),
            jax.ShapeDtypeStruct((E, 1, 1), jnp.float32),
        ),
    )(W2, b2.reshape(E, 1, D), wrT, b_risk.reshape(1, 1))


def _ffn_body(be_ref, valid_ref, xs_ref, w1_ref, b1_ref, w2r_ref, b2r_ref,
              out_ref):
    i = pl.program_id(0)

    @pl.when(valid_ref[0, i] > 0)
    def _():
        xb = xs_ref[...]
        h = jnp.dot(xb, w1_ref[0], preferred_element_type=jnp.float32)
        h = jax.nn.gelu(h + b1_ref[0])
        z = jnp.sum(h * w2r_ref[0], axis=1) + jnp.sum(b2r_ref[...])
        out_ref[...] = z[:, None]


def _ffn(block_expert, valid, xs, W1, b1, w2r, b2r):
    grid_spec = pltpu.PrefetchScalarGridSpec(
        num_scalar_prefetch=2,
        grid=(NB,),
        in_specs=[
            pl.BlockSpec((BLK, D), lambda i, be, va: (i, 0)),
            pl.BlockSpec((1, D, DF), lambda i, be, va: (be[0, i], 0, 0)),
            pl.BlockSpec((1, 1, DF), lambda i, be, va: (be[0, i], 0, 0)),
            pl.BlockSpec((1, 1, DF), lambda i, be, va: (be[0, i], 0, 0)),
            pl.BlockSpec((1, 1, 1), lambda i, be, va: (be[0, i], 0, 0)),
        ],
        out_specs=pl.BlockSpec((BLK, 1), lambda i, be, va: (i, 0)),
    )
    return pl.pallas_call(
        _ffn_body,
        grid_spec=grid_spec,
        out_shape=jax.ShapeDtypeStruct((M, 1), jnp.float32),
    )(block_expert, valid, xs, W1, b1.reshape(E, 1, DF), w2r, b2r)


@functools.partial(
    pl.kernel,
    out_type=jax.ShapeDtypeStruct((N,), jnp.float32),
    mesh=_SC_MESH,
    scratch_types=[
        pltpu.VMEM((TPW,), jnp.int32),
        pltpu.VMEM((TPW,), jnp.int32),
        pltpu.VMEM((TPW,), jnp.float32),
        pltpu.VMEM((TPW,), jnp.float32),
        pltpu.VMEM((TPW,), jnp.float32),
        pltpu.VMEM((TPW,), jnp.float32),
        pltpu.VMEM((TPW,), jnp.float32),
        pltpu.SemaphoreType.DMA,
    ],
)
def _combine(z_hbm, p0_hbm, p1_hbm, g0_hbm, g1_hbm, out_hbm,
             p0_v, p1_v, z0_v, z1_v, g0_v, g1_v, r_v, sem):
    wid = lax.axis_index("s") * 2 + lax.axis_index("c")
    base = wid * TPW
    pltpu.sync_copy(p0_hbm.at[pl.ds(base, TPW)], p0_v)
    pltpu.sync_copy(p1_hbm.at[pl.ds(base, TPW)], p1_v)
    pltpu.sync_copy(g0_hbm.at[pl.ds(base, TPW)], g0_v)
    pltpu.sync_copy(g1_hbm.at[pl.ds(base, TPW)], g1_v)
    pltpu.async_copy(z_hbm.at[p0_v], z0_v, sem).wait()
    pltpu.async_copy(z_hbm.at[p1_v], z1_v, sem).wait()
    for j in range(TPW // 16):
        sl = pl.ds(j * 16, 16)
        r_v[sl] = g0_v[sl] * z0_v[sl] + g1_v[sl] * z1_v[sl]
    pltpu.sync_copy(r_v, out_hbm.at[pl.ds(base, TPW)])


def kernel(x, Wg_group, Wg_expert, W1, b1, W2, b2, W_risk, b_risk):
    pos0, pos1, g0, g1, block_expert, valid = _router(x, Wg_group, Wg_expert)
    p0, p1 = pos0.reshape(N), pos1.reshape(N)
    w2r, b2r = _w2r(W2, b2, W_risk.reshape(1, D), b_risk)
    xs = _dispatch(x, p0, p1)
    z = _ffn(block_expert, valid, xs, W1, b1, w2r, b2r)
    risk = _combine(z.reshape(M), p0, p1, g0.reshape(N), g1.reshape(N))
    return risk
